# baseline (speedup 1.0000x reference)
import jax
import jax.numpy as jnp
from jax.experimental import pallas as pl

_K = 64


def _body(pc_ref, tgt_ref, ox_ref, oy_ref, oz_ref):
    pc = pc_ref[0]
    tgt = tgt_ref[0]
    px = pc[0:1, :]
    py = pc[1:2, :]
    pz = pc[2:3, :]
    npc = px * px + py * py + pz * pz
    nt = jnp.sum(tgt * tgt, axis=1, keepdims=True)
    def r16(x):
        return x.astype(jnp.bfloat16).astype(jnp.float32)
    dot = (r16(tgt[:, 0:1]) * r16(px) + r16(tgt[:, 1:2]) * r16(py)
           + r16(tgt[:, 2:3]) * r16(pz))
    d = nt + npc - 2.0 * dot

    s = jax.lax.bitcast_convert_type(d, jnp.int32)
    int_min = jnp.int32(-(2**31))
    keys = jnp.where(s >= 0, s, jnp.bitwise_xor(jnp.bitwise_not(s), int_min))

    q = keys.shape[0]
    lo0 = jnp.full((q, 1), -(2**31), jnp.int32)
    hi0 = jnp.full((q, 1), 2**31 - 1, jnp.int32)
    tf0 = jnp.full((q, 1), 2**31 - 1, jnp.int32)
    fnd0 = jnp.zeros((q, 1), jnp.int32)

    def cond(st):
        it, lo, hi, tf, fnd = st
        return jnp.logical_and(it < 32, jnp.min(fnd) == 0)

    def step(st):
        it, lo, hi, tf, fnd = st
        for _ in range(8):
            mid = (lo >> 1) + (hi >> 1) + (lo & hi & 1)
            cnt = jnp.sum((keys <= mid).astype(jnp.int32), axis=1,
                          keepdims=True)
            hit = jnp.logical_and(cnt == _K, fnd == 0)
            tf = jnp.where(hit, mid, tf)
            fnd = jnp.where(hit, 1, fnd)
            ge = cnt >= _K
            lo = jnp.where(ge, lo, mid)
            hi = jnp.where(ge, mid, hi)
        return (it + 8, lo, hi, tf, fnd)

    _, _, hi, tf, fnd = jax.lax.while_loop(
        cond, step, (jnp.int32(0), lo0, hi0, tf0, fnd0))
    thr = jnp.where(fnd == 1, tf, hi)
    mask = keys <= thr
    zeros = jnp.zeros_like(d)
    ox_ref[0] = jnp.where(mask, jnp.broadcast_to(px, d.shape), zeros)
    oy_ref[0] = jnp.where(mask, jnp.broadcast_to(py, d.shape), zeros)
    oz_ref[0] = jnp.where(mask, jnp.broadcast_to(pz, d.shape), zeros)


@jax.jit
def kernel(pointcloud, target):
    pc = pointcloud[..., :3]
    b, n, _ = pc.shape
    q = target.shape[1]
    pc_t = jnp.swapaxes(pc, 1, 2)
    plane = jax.ShapeDtypeStruct((b, q, n), jnp.float32)
    ox, oy, oz = pl.pallas_call(
        _body,
        grid=(b,),
        in_specs=[
            pl.BlockSpec((1, 3, n), lambda i: (i, 0, 0)),
            pl.BlockSpec((1, q, 3), lambda i: (i, 0, 0)),
        ],
        out_specs=[pl.BlockSpec((1, q, n), lambda i: (i, 0, 0))] * 3,
        out_shape=[plane] * 3,
    )(pc_t, target)
    return jnp.stack([ox, oy, oz], axis=-1)
